# baseline (speedup 1.0000x reference)
import functools

import jax
import jax.numpy as jnp
import numpy as np
from jax import lax
from jax.experimental import pallas as pl
from jax.experimental.pallas import tpu as pltpu
from jax.experimental.pallas import tpu_sc as plsc

B, S, N, LC = 4, 128, 256, 5
P = LC * (LC + 1) // 2

_NC, _NS, _L = 1, 8, 16
_NW = _NC * _NS
_TOT = B * LC * N
_PER_W = _TOT // _NW
_CHUNKS = _PER_W // _L


_OFF_B = 16
_OFF_LW = 32
_OFF_LB = 32 + P * N
_OFF_LC = 32 + 2 * P * N
_FPACK = _OFF_LC + B * N


def _sc_build_tables(idx_hbm, f_hbm,
                     w_out, ab_out,
                     idx_v, f_v, wv, abv, sem):
    wid = lax.axis_index("s") * _NC + lax.axis_index("c")
    cps = [pltpu.async_copy(idx_hbm.at[wid], idx_v, sem),
           pltpu.async_copy(f_hbm, f_v, sem)]
    for c in cps:
        c.wait()
    for j in range(_CHUNKS):
        sl = pl.ds(j * _L, _L)
        lcidx = idx_v[sl]
        cvec = idx_v[pl.ds(_PER_W + j * _L, _L)]
        ivec = lcidx & (N - 1)
        lcv = plsc.bitcast(plsc.load_gather(f_v, [lcidx + _OFF_LC]),
                           jnp.int32)
        id1 = jnp.maximum(lcv, cvec)
        id2 = jnp.minimum(lcv, cvec)
        kk = lax.shift_right_logical(id1 * (id1 + 1), 1) + id2
        kbase = kk * N + ivec
        aval = plsc.load_gather(f_v, [kk])
        bval = plsc.load_gather(f_v, [kk + _OFF_B])
        lwv = plsc.load_gather(f_v, [kbase + _OFF_LW])
        lbv = plsc.load_gather(f_v, [kbase + _OFF_LB])
        wv[sl] = aval * lwv
        abv[sl] = aval * lbv + bval
    o1 = pltpu.async_copy(wv, w_out.at[wid], sem)
    o2 = pltpu.async_copy(abv, ab_out.at[wid], sem)
    o1.wait()
    o2.wait()


_sc_call = functools.partial(
    pl.kernel,
    mesh=plsc.VectorSubcoreMesh(core_axis_name="c", subcore_axis_name="s",
                                num_cores=_NC, num_subcores=_NS),
    compiler_params=pltpu.CompilerParams(needs_layout_passes=False),
    out_type=[jax.ShapeDtypeStruct((_NW, _PER_W), jnp.float32),
              jax.ShapeDtypeStruct((_NW, _PER_W), jnp.float32)],
    scratch_types=[pltpu.VMEM((2 * _PER_W,), jnp.int32),
                   pltpu.VMEM((_FPACK,), jnp.float32),
                   pltpu.VMEM((_PER_W,), jnp.float32),
                   pltpu.VMEM((_PER_W,), jnp.float32),
                   pltpu.SemaphoreType.DMA],
)(_sc_build_tables)

_T = np.arange(_TOT, dtype=np.int32)
_IDXPACK = jnp.asarray(
    np.stack([((_T // (LC * N)) * N + (_T % N)).reshape(_NW, _PER_W),
              ((_T // N) % LC).reshape(_NW, _PER_W)], axis=1)
    .reshape(_NW, 2 * _PER_W), dtype=jnp.int32)


def _tc_fused(x_ref, w_ref, ab_ref, lc_ref, w2_ref,
              g1_ref, b1_ref, g2_ref, b2_ref, out_ref):
    f32 = jnp.float32
    x = x_ref[:]
    dn = (((1,), (1,)), ((), ()))
    hs = []
    s1 = jnp.zeros((1, N), dtype=f32)
    for b in range(B):
        lcr = lc_ref[b:b + 1, :]
        cio = lax.broadcasted_iota(jnp.int32, (LC, N), 0)
        lcb = jnp.broadcast_to(lcr, (LC, N))
        W = w_ref[b]
        AB = ab_ref[b]
        onesN = jnp.ones((N, 1), dtype=f32)
        Cb = lax.dot_general(AB, onesN, (((1,), (0,)), ((), ())),
                             preferred_element_type=f32)
        H = lax.dot_general(x[b], W, dn, precision=lax.Precision.HIGHEST,
                            preferred_element_type=f32)
        O = (cio == lcb).astype(f32)
        CbO = lax.dot_general(Cb, O, (((0,), (0,)), ((), ())),
                              preferred_element_type=f32)
        hb = lax.dot_general(H, O, (((1,), (0,)), ((), ())),
                             precision=lax.Precision.HIGHEST,
                             preferred_element_type=f32) + CbO
        hs.append(hb)
        s1 = s1 + jnp.sum(hb, axis=0, keepdims=True)

    inv_bs = 1.0 / (B * S)
    mean1 = s1 * inv_bs
    ss = jnp.zeros((1, N), dtype=f32)
    for b in range(B):
        d = hs[b] - mean1
        ss = ss + jnp.sum(d * d, axis=0, keepdims=True)
    rstd1 = lax.rsqrt(ss * inv_bs + 1e-5)
    scale1 = rstd1 * g1_ref[:]
    shift1 = b1_ref[:] - mean1 * scale1

    obs = []
    s2 = jnp.zeros((S, 1), dtype=f32)
    w2 = w2_ref[:]
    for b in range(B):
        g = jnp.maximum(hs[b] * scale1 + shift1 + x[b], 0.0)
        ob = lax.dot_general(w2, g, (((1,), (0,)), ((), ())),
                             precision=lax.Precision.HIGHEST,
                             preferred_element_type=f32)
        obs.append(ob)
        s2 = s2 + jnp.sum(ob, axis=1, keepdims=True)

    inv_bn = 1.0 / (B * N)
    mean2 = s2 * inv_bn
    ss2 = jnp.zeros((S, 1), dtype=f32)
    for b in range(B):
        d = obs[b] - mean2
        ss2 = ss2 + jnp.sum(d * d, axis=1, keepdims=True)
    rstd2 = lax.rsqrt(ss2 * inv_bn + 1e-5)
    scale2 = rstd2 * g2_ref[:]
    shift2 = b2_ref[:] - mean2 * scale2
    for b in range(B):
        out_ref[b] = obs[b] * scale2 + shift2


def kernel(x, lc, edgeCalPara, lcconv_w, lcconv_b, conv2_w,
           bn1_gamma, bn1_beta, bn2_gamma, bn2_beta):
    a_vec = jnp.pad(edgeCalPara[0, :, 0], (0, 16 - P))
    b_vec = jnp.pad(edgeCalPara[1, :, 0], (0, 16 - P))
    fpack = jnp.concatenate([a_vec, b_vec, lcconv_w.reshape(P * N),
                             lcconv_b.reshape(P * N),
                             lax.bitcast_convert_type(lc.reshape(B * N),
                                                      jnp.float32)])
    w_flat, ab_flat = _sc_call(_IDXPACK, fpack)
    W = w_flat.reshape(B, LC, N)
    AB = ab_flat.reshape(B, LC, N)
    out = pl.pallas_call(
        _tc_fused,
        out_shape=jax.ShapeDtypeStruct((B, S, N), jnp.float32),
    )(x, W, AB, lc, conv2_w,
      bn1_gamma.reshape(1, N), bn1_beta.reshape(1, N),
      bn2_gamma.reshape(S, 1), bn2_beta.reshape(S, 1))
    return (out, lc)


# speedup vs baseline: 1.0224x; 1.0224x over previous
import functools

import jax
import jax.numpy as jnp
import numpy as np
from jax import lax
from jax.experimental import pallas as pl
from jax.experimental.pallas import tpu as pltpu
from jax.experimental.pallas import tpu_sc as plsc

B, S, N, LC = 4, 128, 256, 5
P = LC * (LC + 1) // 2

_NC, _NS, _L = 1, 16, 16
_NW = _NC * _NS
_TOT = B * LC * N
_PER_W = _TOT // _NW
_CHUNKS = _PER_W // _L


_OFF_B = 16
_OFF_LW = 32
_OFF_LB = 32 + P * N
_OFF_LC = 32 + 2 * P * N
_FPACK = _OFF_LC + B * N


def _sc_build_tables(idx_hbm, f_hbm,
                     w_out, ab_out,
                     idx_v, f_v, wv, abv, sem):
    wid = lax.axis_index("s") * _NC + lax.axis_index("c")
    cps = [pltpu.async_copy(idx_hbm.at[wid], idx_v, sem),
           pltpu.async_copy(f_hbm, f_v, sem)]
    for c in cps:
        c.wait()
    for j in range(_CHUNKS):
        sl = pl.ds(j * _L, _L)
        lcidx = idx_v[sl]
        cvec = idx_v[pl.ds(_PER_W + j * _L, _L)]
        ivec = lcidx & (N - 1)
        lcv = plsc.bitcast(plsc.load_gather(f_v, [lcidx + _OFF_LC]),
                           jnp.int32)
        id1 = jnp.maximum(lcv, cvec)
        id2 = jnp.minimum(lcv, cvec)
        kk = lax.shift_right_logical(id1 * (id1 + 1), 1) + id2
        kbase = kk * N + ivec
        aval = plsc.load_gather(f_v, [kk])
        bval = plsc.load_gather(f_v, [kk + _OFF_B])
        lwv = plsc.load_gather(f_v, [kbase + _OFF_LW])
        lbv = plsc.load_gather(f_v, [kbase + _OFF_LB])
        wv[sl] = aval * lwv
        abv[sl] = aval * lbv + bval
    o1 = pltpu.async_copy(wv, w_out.at[wid], sem)
    o2 = pltpu.async_copy(abv, ab_out.at[wid], sem)
    o1.wait()
    o2.wait()


_sc_call = functools.partial(
    pl.kernel,
    mesh=plsc.VectorSubcoreMesh(core_axis_name="c", subcore_axis_name="s",
                                num_cores=_NC, num_subcores=_NS),
    compiler_params=pltpu.CompilerParams(needs_layout_passes=False,
                                         disable_bounds_checks=True),
    out_type=[jax.ShapeDtypeStruct((_NW, _PER_W), jnp.float32),
              jax.ShapeDtypeStruct((_NW, _PER_W), jnp.float32)],
    scratch_types=[pltpu.VMEM((2 * _PER_W,), jnp.int32),
                   pltpu.VMEM((_FPACK,), jnp.float32),
                   pltpu.VMEM((_PER_W,), jnp.float32),
                   pltpu.VMEM((_PER_W,), jnp.float32),
                   pltpu.SemaphoreType.DMA],
)(_sc_build_tables)

_T = np.arange(_TOT, dtype=np.int32)
_IDXPACK = jnp.asarray(
    np.stack([((_T // (LC * N)) * N + (_T % N)).reshape(_NW, _PER_W),
              ((_T // N) % LC).reshape(_NW, _PER_W)], axis=1)
    .reshape(_NW, 2 * _PER_W), dtype=jnp.int32)


def _tc_fused(x_ref, w_ref, ab_ref, lc_ref, w2_ref,
              g1_ref, b1_ref, g2_ref, b2_ref, out_ref):
    f32 = jnp.float32
    x = x_ref[:]
    dn = (((1,), (1,)), ((), ()))
    hs = []
    s1 = jnp.zeros((1, N), dtype=f32)
    for b in range(B):
        lcr = lc_ref[b:b + 1, :]
        cio = lax.broadcasted_iota(jnp.int32, (LC, N), 0)
        lcb = jnp.broadcast_to(lcr, (LC, N))
        W = w_ref[b]
        AB = ab_ref[b]
        onesN = jnp.ones((N, 1), dtype=f32)
        Cb = lax.dot_general(AB, onesN, (((1,), (0,)), ((), ())),
                             preferred_element_type=f32)
        H = lax.dot_general(x[b], W, dn, precision=lax.Precision.HIGHEST,
                            preferred_element_type=f32)
        O = (cio == lcb).astype(f32)
        CbO = lax.dot_general(Cb, O, (((0,), (0,)), ((), ())),
                              preferred_element_type=f32)
        hb = lax.dot_general(H, O, (((1,), (0,)), ((), ())),
                             precision=lax.Precision.HIGHEST,
                             preferred_element_type=f32) + CbO
        hs.append(hb)
        s1 = s1 + jnp.sum(hb, axis=0, keepdims=True)

    inv_bs = 1.0 / (B * S)
    mean1 = s1 * inv_bs
    ss = jnp.zeros((1, N), dtype=f32)
    for b in range(B):
        d = hs[b] - mean1
        ss = ss + jnp.sum(d * d, axis=0, keepdims=True)
    rstd1 = lax.rsqrt(ss * inv_bs + 1e-5)
    scale1 = rstd1 * g1_ref[:]
    shift1 = b1_ref[:] - mean1 * scale1

    obs = []
    s2 = jnp.zeros((S, 1), dtype=f32)
    w2 = w2_ref[:]
    for b in range(B):
        g = jnp.maximum(hs[b] * scale1 + shift1 + x[b], 0.0)
        ob = lax.dot_general(w2, g, (((1,), (0,)), ((), ())),
                             precision=lax.Precision.HIGHEST,
                             preferred_element_type=f32)
        obs.append(ob)
        s2 = s2 + jnp.sum(ob, axis=1, keepdims=True)

    inv_bn = 1.0 / (B * N)
    mean2 = s2 * inv_bn
    ss2 = jnp.zeros((S, 1), dtype=f32)
    for b in range(B):
        d = obs[b] - mean2
        ss2 = ss2 + jnp.sum(d * d, axis=1, keepdims=True)
    rstd2 = lax.rsqrt(ss2 * inv_bn + 1e-5)
    scale2 = rstd2 * g2_ref[:]
    shift2 = b2_ref[:] - mean2 * scale2
    for b in range(B):
        out_ref[b] = obs[b] * scale2 + shift2


def kernel(x, lc, edgeCalPara, lcconv_w, lcconv_b, conv2_w,
           bn1_gamma, bn1_beta, bn2_gamma, bn2_beta):
    a_vec = jnp.pad(edgeCalPara[0, :, 0], (0, 16 - P))
    b_vec = jnp.pad(edgeCalPara[1, :, 0], (0, 16 - P))
    fpack = jnp.concatenate([a_vec, b_vec, lcconv_w.reshape(P * N),
                             lcconv_b.reshape(P * N),
                             lax.bitcast_convert_type(lc.reshape(B * N),
                                                      jnp.float32)])
    w_flat, ab_flat = _sc_call(_IDXPACK, fpack)
    W = w_flat.reshape(B, LC, N)
    AB = ab_flat.reshape(B, LC, N)
    out = pl.pallas_call(
        _tc_fused,
        out_shape=jax.ShapeDtypeStruct((B, S, N), jnp.float32),
    )(x, W, AB, lc, conv2_w,
      bn1_gamma.reshape(1, N), bn1_beta.reshape(1, N),
      bn2_gamma.reshape(S, 1), bn2_beta.reshape(S, 1))
    return (out, lc)


# speedup vs baseline: 1.1114x; 1.0870x over previous
import functools

import jax
import jax.numpy as jnp
import numpy as np
from jax import lax
from jax.experimental import pallas as pl
from jax.experimental.pallas import tpu as pltpu
from jax.experimental.pallas import tpu_sc as plsc

B, S, N, LC = 4, 128, 256, 5
P = LC * (LC + 1) // 2

_NC, _NS, _L = 1, 16, 16
_NW = _NC * _NS
_TOT = B * LC * N
_PER_W = _TOT // _NW
_CHUNKS = _PER_W // _L


_OFF_B = 16
_OFF_LW = 32
_OFF_LB = 32 + P * N
_OFF_LC = 32 + 2 * P * N
_FPACK = _OFF_LC + B * N


def _sc_build_tables(idx_hbm, f_hbm,
                     w_out,
                     idx_v, f_v, wv, sem):
    wid = lax.axis_index("s") * _NC + lax.axis_index("c")
    cps = [pltpu.async_copy(idx_hbm.at[wid], idx_v, sem),
           pltpu.async_copy(f_hbm, f_v, sem)]
    for c in cps:
        c.wait()
    for j in range(_CHUNKS):
        sl = pl.ds(j * _L, _L)
        lcidx = idx_v[sl]
        cvec = idx_v[pl.ds(_PER_W + j * _L, _L)]
        ivec = lcidx & (N - 1)
        lcv = plsc.bitcast(plsc.load_gather(f_v, [lcidx + _OFF_LC]),
                           jnp.int32)
        id1 = jnp.maximum(lcv, cvec)
        id2 = jnp.minimum(lcv, cvec)
        kk = lax.shift_right_logical(id1 * (id1 + 1), 1) + id2
        kbase = kk * N + ivec
        aval = plsc.load_gather(f_v, [kk])
        bval = plsc.load_gather(f_v, [kk + _OFF_B])
        lwv = plsc.load_gather(f_v, [kbase + _OFF_LW])
        lbv = plsc.load_gather(f_v, [kbase + _OFF_LB])
        wv[sl] = aval * lwv
        wv[pl.ds(_PER_W + j * _L, _L)] = aval * lbv + bval
    pltpu.async_copy(wv, w_out.at[wid], sem).wait()


_sc_call = functools.partial(
    pl.kernel,
    mesh=plsc.VectorSubcoreMesh(core_axis_name="c", subcore_axis_name="s",
                                num_cores=_NC, num_subcores=_NS),
    compiler_params=pltpu.CompilerParams(needs_layout_passes=False,
                                         disable_bounds_checks=True),
    out_type=jax.ShapeDtypeStruct((_NW, 2 * _PER_W), jnp.float32),
    scratch_types=[pltpu.VMEM((2 * _PER_W,), jnp.int32),
                   pltpu.VMEM((_FPACK,), jnp.float32),
                   pltpu.VMEM((2 * _PER_W,), jnp.float32),
                   pltpu.SemaphoreType.DMA],
)(_sc_build_tables)

_T = np.arange(_TOT, dtype=np.int32)
_IDXPACK = jnp.asarray(
    np.stack([((_T // (LC * N)) * N + (_T % N)).reshape(_NW, _PER_W),
              ((_T // N) % LC).reshape(_NW, _PER_W)], axis=1)
    .reshape(_NW, 2 * _PER_W), dtype=jnp.int32)


def _unpack_rows(pk_ref, b, off):
    rows = []
    for c in range(LC):
        o0 = b * LC * N + c * N
        r, c0 = o0 // _PER_W, o0 % _PER_W
        take = min(_PER_W - c0, N)
        parts = [pk_ref[r:r + 1, off + c0:off + c0 + take]]
        if take < N:
            parts.append(pk_ref[r + 1:r + 2, off:off + N - take])
        rows.append(jnp.concatenate(parts, axis=1) if len(parts) > 1
                    else parts[0])
    return jnp.concatenate(rows, axis=0)


def _tc_fused(x_ref, pk_ref, lc_ref, w2_ref,
              g1_ref, b1_ref, g2_ref, b2_ref, out_ref):
    f32 = jnp.float32
    x = x_ref[:]
    dn = (((1,), (1,)), ((), ()))
    hs = []
    s1 = jnp.zeros((1, N), dtype=f32)
    for b in range(B):
        lcr = lc_ref[b:b + 1, :]
        cio = lax.broadcasted_iota(jnp.int32, (LC, N), 0)
        lcb = jnp.broadcast_to(lcr, (LC, N))
        W = _unpack_rows(pk_ref, b, 0)
        AB = _unpack_rows(pk_ref, b, _PER_W)
        onesN = jnp.ones((N, 1), dtype=f32)
        Cb = lax.dot_general(AB, onesN, (((1,), (0,)), ((), ())),
                             preferred_element_type=f32)
        H = lax.dot_general(x[b], W, dn, precision=lax.Precision.HIGHEST,
                            preferred_element_type=f32)
        O = (cio == lcb).astype(f32)
        CbO = lax.dot_general(Cb, O, (((0,), (0,)), ((), ())),
                              preferred_element_type=f32)
        hb = lax.dot_general(H, O, (((1,), (0,)), ((), ())),
                             precision=lax.Precision.HIGHEST,
                             preferred_element_type=f32) + CbO
        hs.append(hb)
        s1 = s1 + jnp.sum(hb, axis=0, keepdims=True)

    inv_bs = 1.0 / (B * S)
    mean1 = s1 * inv_bs
    ss = jnp.zeros((1, N), dtype=f32)
    for b in range(B):
        d = hs[b] - mean1
        ss = ss + jnp.sum(d * d, axis=0, keepdims=True)
    rstd1 = lax.rsqrt(ss * inv_bs + 1e-5)
    scale1 = rstd1 * g1_ref[:]
    shift1 = b1_ref[:] - mean1 * scale1

    obs = []
    s2 = jnp.zeros((S, 1), dtype=f32)
    w2 = w2_ref[:]
    for b in range(B):
        g = jnp.maximum(hs[b] * scale1 + shift1 + x[b], 0.0)
        ob = lax.dot_general(w2, g, (((1,), (0,)), ((), ())),
                             precision=lax.Precision.HIGHEST,
                             preferred_element_type=f32)
        obs.append(ob)
        s2 = s2 + jnp.sum(ob, axis=1, keepdims=True)

    inv_bn = 1.0 / (B * N)
    mean2 = s2 * inv_bn
    ss2 = jnp.zeros((S, 1), dtype=f32)
    for b in range(B):
        d = obs[b] - mean2
        ss2 = ss2 + jnp.sum(d * d, axis=1, keepdims=True)
    rstd2 = lax.rsqrt(ss2 * inv_bn + 1e-5)
    scale2 = rstd2 * g2_ref[:]
    shift2 = b2_ref[:] - mean2 * scale2
    for b in range(B):
        out_ref[b] = obs[b] * scale2 + shift2


def kernel(x, lc, edgeCalPara, lcconv_w, lcconv_b, conv2_w,
           bn1_gamma, bn1_beta, bn2_gamma, bn2_beta):
    a_vec = jnp.pad(edgeCalPara[0, :, 0], (0, 16 - P))
    b_vec = jnp.pad(edgeCalPara[1, :, 0], (0, 16 - P))
    fpack = jnp.concatenate([a_vec, b_vec, lcconv_w.reshape(P * N),
                             lcconv_b.reshape(P * N),
                             lax.bitcast_convert_type(lc.reshape(B * N),
                                                      jnp.float32)])
    packed = _sc_call(_IDXPACK, fpack)
    out = pl.pallas_call(
        _tc_fused,
        out_shape=jax.ShapeDtypeStruct((B, S, N), jnp.float32),
    )(x, packed, lc, conv2_w,
      bn1_gamma.reshape(1, N), bn1_beta.reshape(1, N),
      bn2_gamma.reshape(S, 1), bn2_beta.reshape(S, 1))
    return (out, lc)
